# baseline (speedup 1.0000x reference)
import functools

import jax
import jax.numpy as jnp
from jax import lax
from jax.experimental import pallas as pl
from jax.experimental.pallas import tpu as pltpu
from jax.experimental.pallas import tpu_sc as plsc

_VOCAB = 1000
_D = 1000
_B = 1024
_T = 50
_NTOK = _B * _T

_NC = 2
_NS = 16
_NW = _NC * _NS
_TPW = _NTOK // _NW

_CH = 32
_NCH = _TPW // _CH
_NBUF = 4
_LOOK = 2


def _body(table_hbm, idx_hbm, out_hbm, idx_v, bufs, g0, g1, g2, g3,
          s0, s1, s2, s3):
    wid = lax.axis_index("s") * _NC + lax.axis_index("c")
    base = wid * _TPW
    pltpu.sync_copy(idx_hbm.at[pl.ds(base, _TPW)], idx_v)

    gsem = (g0, g1, g2, g3)
    ssem = (s0, s1, s2, s3)

    def gather_dma(c, b):
        return pltpu.make_async_copy(
            table_hbm.at[idx_v.at[pl.ds(pl.multiple_of(c * _CH, 8), _CH)]],
            bufs.at[b],
            gsem[b],
        )

    def scatter_dma(c, b):
        return pltpu.make_async_copy(
            bufs.at[b],
            out_hbm.at[pl.ds(base + c * _CH, _CH)],
            ssem[b],
        )

    for b in range(_LOOK):
        gather_dma(b, b).start()

    _NFULL = _NCH // _NBUF

    def outer(i, carry):
        cc = i * _NBUF
        for b in range(_NBUF):
            c = cc + b
            gather_dma(c, b).wait()
            scatter_dma(c, b).start()

            bp = (b + _LOOK) % _NBUF

            @pl.when(c >= _LOOK)
            def _():
                scatter_dma(c - _LOOK, bp).wait()

            gather_dma(c + _LOOK, bp).start()

        return carry

    lax.fori_loop(0, _NFULL, outer, 0)

    for c in range(_NFULL * _NBUF, _NCH):
        b = c % _NBUF
        gather_dma(c, b).wait()
        scatter_dma(c, b).start()
        scatter_dma(c - _LOOK, (b + _LOOK) % _NBUF).wait()
    for c in range(_NCH - _LOOK, _NCH):
        scatter_dma(c, c % _NBUF).wait()


@functools.partial(
    pl.kernel,
    mesh=plsc.VectorSubcoreMesh(core_axis_name="c", subcore_axis_name="s"),
    compiler_params=pltpu.CompilerParams(use_tc_tiling_on_sc=False),
    out_type=jax.ShapeDtypeStruct((_NTOK, _D), jnp.float32),
    scratch_types=[
        pltpu.VMEM((_TPW,), jnp.int32),
        pltpu.VMEM((_NBUF, _CH, _D), jnp.float32),
    ] + [pltpu.SemaphoreType.DMA] * (2 * _NBUF),
)
def _gather_rows(table_hbm, idx_hbm, out_hbm, idx_v, bufs, *sems):
    _body(table_hbm, idx_hbm, out_hbm, idx_v, bufs, *sems)


def kernel(idx, table):
    flat = idx.reshape(_NTOK)
    out = _gather_rows(table, flat)
    return out.reshape(_B, _T, _D)
